# Initial kernel scaffold; baseline (speedup 1.0000x reference)
import jax
import jax.numpy as jnp
from jax.experimental import pallas as pl


def kernel(x):
    raise NotImplementedError("write your pallas kernel here")


# speedup vs baseline: 3.8984x; 3.8984x over previous
import functools

import jax
import jax.numpy as jnp
from jax.experimental import pallas as pl

_K = 64
_ITERS = 10
_D = 5
_DP = 8
_HIGHEST = jax.lax.Precision.HIGHEST


def _kmeans_body(xt_ref, xnk_ref, c0_ref, out_ref, *, n_chunks, chunk):
    feat_mask = (jax.lax.broadcasted_iota(jnp.int32, (1, _DP), 1) < _D
                 ).astype(jnp.float32)
    col5 = (jax.lax.broadcasted_iota(jnp.int32, (1, _DP), 1) == _D
            ).astype(jnp.float32)

    def make_caug(c):
        cnorm = jnp.sum(c * c, axis=1, keepdims=True)
        return -2.0 * c + cnorm * col5

    def labels_for_chunk(c_aug, i):
        xt_ch = xt_ref[0, :, pl.ds(i * chunk, chunk)]
        scores = jax.lax.dot_general(
            c_aug, xt_ch, (((1,), (0,)), ((), ())),
            precision=_HIGHEST, preferred_element_type=jnp.float32)
        mins = jnp.min(scores, axis=0, keepdims=True)
        kiota = jax.lax.broadcasted_iota(jnp.int32, (_K, chunk), 0)
        lab = jnp.min(jnp.where(scores == mins, kiota, _K),
                      axis=0, keepdims=True)
        return lab, kiota

    def update_iter(_, c):
        c_aug = make_caug(c)

        def chunk_body(i, sums):
            lab, kiota = labels_for_chunk(c_aug, i)
            oh = (kiota == lab).astype(jnp.float32)
            xnk_ch = xnk_ref[0, pl.ds(i * chunk, chunk), :]
            return sums + jax.lax.dot_general(
                oh, xnk_ch, (((1,), (0,)), ((), ())),
                preferred_element_type=jnp.float32)

        sums = jax.lax.fori_loop(0, n_chunks, chunk_body,
                                 jnp.zeros((_K, _DP), jnp.float32))
        counts = sums[:, _D:_D + 1]
        return (sums / jnp.maximum(counts, 1.0)) * feat_mask

    c = jax.lax.fori_loop(0, _ITERS, update_iter, c0_ref[0])

    c_aug = make_caug(c)

    def final_chunk(i, _):
        lab, _kiota = labels_for_chunk(c_aug, i)
        out_ref[0, :, pl.ds(i * chunk, chunk)] = lab
        return 0

    jax.lax.fori_loop(0, n_chunks, final_chunk, 0)


@jax.jit
def kernel(x):
    x = x.astype(jnp.float32) / 255.0
    x = 2.0 * x - 1.0
    b, c, h, w = x.shape
    n = h * w
    ys, xs = jnp.meshgrid(jnp.arange(h, dtype=jnp.float32),
                          jnp.arange(w, dtype=jnp.float32), indexing='ij')
    xgrid = 2.0 * xs / (w - 1) - 1.0
    ygrid = 2.0 * ys / (h - 1) - 1.0
    coords = jnp.broadcast_to(jnp.stack([ygrid, xgrid], axis=0)[None],
                              (b, 2, h, w))
    feats = jnp.concatenate([x, coords], axis=1).reshape(b, _D, n)

    xt = jnp.concatenate([feats,
                          jnp.ones((b, 1, n), jnp.float32),
                          jnp.zeros((b, 2, n), jnp.float32)], axis=1)
    xnk = xt.transpose(0, 2, 1)

    k_inds = jax.random.randint(jax.random.key(1), (b, _K), 0, n)
    c0 = jnp.take_along_axis(feats, k_inds[:, None, :], axis=2)
    c0 = c0.transpose(0, 2, 1)
    c0 = jnp.concatenate([c0, jnp.zeros((b, _K, _DP - _D), jnp.float32)],
                         axis=2)

    n_chunks = 8
    chunk = n // n_chunks

    labels = pl.pallas_call(
        functools.partial(_kmeans_body, n_chunks=n_chunks, chunk=chunk),
        grid=(b,),
        in_specs=[
            pl.BlockSpec((1, _DP, n), lambda i: (i, 0, 0)),
            pl.BlockSpec((1, n, _DP), lambda i: (i, 0, 0)),
            pl.BlockSpec((1, _K, _DP), lambda i: (i, 0, 0)),
        ],
        out_specs=pl.BlockSpec((1, 1, n), lambda i: (i, 0, 0)),
        out_shape=jax.ShapeDtypeStruct((b, 1, n), jnp.int32),
    )(xt, xnk, c0)

    return labels.reshape(b, h, w)


# speedup vs baseline: 5.3477x; 1.3718x over previous
import functools

import jax
import jax.numpy as jnp
from jax.experimental import pallas as pl
from jax.experimental.pallas import tpu as pltpu

_K = 64
_ITERS = 10
_D = 5
_DP = 8


def _kmeans_body(xt_ref, c0_ref, xnk_hbm, out_ref, xcat_scr, xnk_scr, dma_sem,
                 *, n_chunks, chunk):
    copy = pltpu.make_async_copy(xnk_hbm.at[pl.program_id(0)], xnk_scr,
                                 dma_sem)
    copy.start()

    def build_chunk(i, _):
        sl = pl.ds(i * chunk, chunk)
        x_ch = xt_ref[0, :, sl]
        hi = x_ch.astype(jnp.bfloat16)
        r1 = x_ch - hi.astype(jnp.float32)
        mid = r1.astype(jnp.bfloat16)
        lo = (r1 - mid.astype(jnp.float32)).astype(jnp.bfloat16)
        xcat_scr[pl.ds(0, _DP), sl] = hi
        xcat_scr[pl.ds(_DP, _DP), sl] = mid
        xcat_scr[pl.ds(2 * _DP, _DP), sl] = hi
        xcat_scr[pl.ds(3 * _DP, _DP), sl] = lo
        xcat_scr[pl.ds(4 * _DP, _DP), sl] = mid
        xcat_scr[pl.ds(5 * _DP, _DP), sl] = hi
        return 0

    jax.lax.fori_loop(0, n_chunks, build_chunk, 0)

    feat_mask = (jax.lax.broadcasted_iota(jnp.int32, (1, _DP), 1) < _D
                 ).astype(jnp.float32)
    col5 = (jax.lax.broadcasted_iota(jnp.int32, (1, _DP), 1) == _D
            ).astype(jnp.float32)

    def make_ccat(c):
        cnorm = jnp.sum(c * c, axis=1, keepdims=True)
        c_aug = -2.0 * c + cnorm * col5
        hi = c_aug.astype(jnp.bfloat16)
        r1 = c_aug - hi.astype(jnp.float32)
        mid = r1.astype(jnp.bfloat16)
        lo = (r1 - mid.astype(jnp.float32)).astype(jnp.bfloat16)
        return jnp.concatenate([hi, hi, mid, hi, mid, lo], axis=1)

    def labels_for_chunk(c_cat, i):
        x_ch = xcat_scr[:, pl.ds(i * chunk, chunk)]
        scores = jax.lax.dot_general(
            c_cat, x_ch, (((1,), (0,)), ((), ())),
            preferred_element_type=jnp.float32)
        mins = jnp.min(scores, axis=0, keepdims=True)
        kiota = jax.lax.broadcasted_iota(jnp.int32, (_K, chunk), 0)
        lab = jnp.min(jnp.where(scores == mins, kiota, _K),
                      axis=0, keepdims=True)
        return lab, kiota

    def update_iter(_, c):
        c_cat = make_ccat(c)

        def chunk_body(i, sums):
            lab, kiota = labels_for_chunk(c_cat, i)
            oh = (kiota == lab).astype(jnp.float32)
            xnk_ch = xnk_scr[pl.ds(i * chunk, chunk), :]
            return sums + jax.lax.dot_general(
                oh, xnk_ch, (((1,), (0,)), ((), ())),
                preferred_element_type=jnp.float32)

        sums = jax.lax.fori_loop(0, n_chunks, chunk_body,
                                 jnp.zeros((_K, _DP), jnp.float32))
        counts = sums[:, _D:_D + 1]
        return (sums / jnp.maximum(counts, 1.0)) * feat_mask

    copy.wait()
    c = jax.lax.fori_loop(0, _ITERS, update_iter, c0_ref[0])

    c_cat = make_ccat(c)

    def final_chunk(i, _):
        lab, _kiota = labels_for_chunk(c_cat, i)
        out_ref[0, :, pl.ds(i * chunk, chunk)] = lab
        return 0

    jax.lax.fori_loop(0, n_chunks, final_chunk, 0)


@jax.jit
def kernel(x):
    x = x.astype(jnp.float32) / 255.0
    x = 2.0 * x - 1.0
    b, c, h, w = x.shape
    n = h * w
    ys, xs = jnp.meshgrid(jnp.arange(h, dtype=jnp.float32),
                          jnp.arange(w, dtype=jnp.float32), indexing='ij')
    xgrid = 2.0 * xs / (w - 1) - 1.0
    ygrid = 2.0 * ys / (h - 1) - 1.0
    coords = jnp.broadcast_to(jnp.stack([ygrid, xgrid], axis=0)[None],
                              (b, 2, h, w))
    feats = jnp.concatenate([x, coords], axis=1).reshape(b, _D, n)

    xt = jnp.concatenate([feats,
                          jnp.ones((b, 1, n), jnp.float32),
                          jnp.zeros((b, 2, n), jnp.float32)], axis=1)
    xnk = xt.transpose(0, 2, 1)

    k_inds = jax.random.randint(jax.random.key(1), (b, _K), 0, n)
    c0 = jnp.take_along_axis(feats, k_inds[:, None, :], axis=2)
    c0 = c0.transpose(0, 2, 1)
    c0 = jnp.concatenate([c0, jnp.zeros((b, _K, _DP - _D), jnp.float32)],
                         axis=2)

    n_chunks = 8
    chunk = n // n_chunks

    labels = pl.pallas_call(
        functools.partial(_kmeans_body, n_chunks=n_chunks, chunk=chunk),
        grid=(b,),
        in_specs=[
            pl.BlockSpec((1, _DP, n), lambda i: (i, 0, 0)),
            pl.BlockSpec((1, _K, _DP), lambda i: (i, 0, 0)),
            pl.BlockSpec(memory_space=pltpu.MemorySpace.HBM),
        ],
        out_specs=pl.BlockSpec((1, 1, n), lambda i: (i, 0, 0)),
        out_shape=jax.ShapeDtypeStruct((b, 1, n), jnp.int32),
        scratch_shapes=[
            pltpu.MemorySpace.VMEM((6 * _DP, n), jnp.bfloat16),
            pltpu.MemorySpace.VMEM((n, _DP), jnp.float32),
            pltpu.SemaphoreType.DMA,
        ],
        compiler_params=pltpu.CompilerParams(
            dimension_semantics=("arbitrary",)),
    )(xt, c0, xnk)

    return labels.reshape(b, h, w)


# speedup vs baseline: 5.3548x; 1.0013x over previous
import functools

import jax
import jax.numpy as jnp
from jax.experimental import pallas as pl
from jax.experimental.pallas import tpu as pltpu

_K = 64
_ITERS = 10
_D = 5
_DP = 8


def _kmeans_body(xt_ref, c0_ref, xnk_hbm, out_ref, xcat_scr, xnk_scr, dma_sem,
                 *, n_chunks, chunk):
    copy = pltpu.make_async_copy(xnk_hbm.at[pl.program_id(0)], xnk_scr,
                                 dma_sem)
    copy.start()

    def build_chunk(i, _):
        sl = pl.ds(i * chunk, chunk)
        x_ch = xt_ref[0, :, sl]
        hi = x_ch.astype(jnp.bfloat16)
        r1 = x_ch - hi.astype(jnp.float32)
        mid = r1.astype(jnp.bfloat16)
        lo = (r1 - mid.astype(jnp.float32)).astype(jnp.bfloat16)
        xcat_scr[pl.ds(0, _DP), sl] = hi
        xcat_scr[pl.ds(_DP, _DP), sl] = mid
        xcat_scr[pl.ds(2 * _DP, _DP), sl] = hi
        xcat_scr[pl.ds(3 * _DP, _DP), sl] = lo
        xcat_scr[pl.ds(4 * _DP, _DP), sl] = mid
        xcat_scr[pl.ds(5 * _DP, _DP), sl] = hi
        return 0

    jax.lax.fori_loop(0, n_chunks, build_chunk, 0)

    feat_mask = (jax.lax.broadcasted_iota(jnp.int32, (1, _DP), 1) < _D
                 ).astype(jnp.float32)
    col5 = (jax.lax.broadcasted_iota(jnp.int32, (1, _DP), 1) == _D
            ).astype(jnp.float32)

    def make_ccat(c):
        cnorm = jnp.sum(c * c, axis=1, keepdims=True)
        c_aug = -2.0 * c + cnorm * col5
        hi = c_aug.astype(jnp.bfloat16)
        r1 = c_aug - hi.astype(jnp.float32)
        mid = r1.astype(jnp.bfloat16)
        lo = (r1 - mid.astype(jnp.float32)).astype(jnp.bfloat16)
        return jnp.concatenate([hi, hi, mid, hi, mid, lo], axis=1)

    def labels_for_chunk(c_cat, i):
        x_ch = xcat_scr[:, pl.ds(i * chunk, chunk)]
        scores = jax.lax.dot_general(
            c_cat, x_ch, (((1,), (0,)), ((), ())),
            preferred_element_type=jnp.float32)
        mins = jnp.min(scores, axis=0, keepdims=True)
        kiota = jax.lax.broadcasted_iota(jnp.int32, (_K, chunk), 0)
        lab = jnp.min(jnp.where(scores == mins, kiota, _K),
                      axis=0, keepdims=True)
        return lab, kiota

    def update_iter(_, c):
        c_cat = make_ccat(c)

        def chunk_body(i, sums):
            lab, kiota = labels_for_chunk(c_cat, i)
            oh = (kiota == lab).astype(jnp.float32)
            xnk_ch = xnk_scr[pl.ds(i * chunk, chunk), :]
            return sums + jax.lax.dot_general(
                oh, xnk_ch, (((1,), (0,)), ((), ())),
                preferred_element_type=jnp.float32)

        sums = jax.lax.fori_loop(0, n_chunks, chunk_body,
                                 jnp.zeros((_K, _DP), jnp.float32))
        counts = sums[:, _D:_D + 1]
        return (sums / jnp.maximum(counts, 1.0)) * feat_mask

    copy.wait()
    c = jax.lax.fori_loop(0, _ITERS, update_iter, c0_ref[0])

    c_cat = make_ccat(c)

    def final_chunk(i, _):
        lab, _kiota = labels_for_chunk(c_cat, i)
        out_ref[0, :, pl.ds(i * chunk, chunk)] = lab
        return 0

    jax.lax.fori_loop(0, n_chunks, final_chunk, 0)


@jax.jit
def kernel(x):
    x = x.astype(jnp.float32) / 255.0
    x = 2.0 * x - 1.0
    b, c, h, w = x.shape
    n = h * w
    ys, xs = jnp.meshgrid(jnp.arange(h, dtype=jnp.float32),
                          jnp.arange(w, dtype=jnp.float32), indexing='ij')
    xgrid = 2.0 * xs / (w - 1) - 1.0
    ygrid = 2.0 * ys / (h - 1) - 1.0
    coords = jnp.broadcast_to(jnp.stack([ygrid, xgrid], axis=0)[None],
                              (b, 2, h, w))
    feats = jnp.concatenate([x, coords], axis=1).reshape(b, _D, n)

    xt = jnp.concatenate([feats,
                          jnp.ones((b, 1, n), jnp.float32),
                          jnp.zeros((b, 2, n), jnp.float32)], axis=1)
    xnk = xt.transpose(0, 2, 1)

    k_inds = jax.random.randint(jax.random.key(1), (b, _K), 0, n)
    c0 = jnp.take_along_axis(feats, k_inds[:, None, :], axis=2)
    c0 = c0.transpose(0, 2, 1)
    c0 = jnp.concatenate([c0, jnp.zeros((b, _K, _DP - _D), jnp.float32)],
                         axis=2)

    n_chunks = 8
    chunk = n // n_chunks

    labels = pl.pallas_call(
        functools.partial(_kmeans_body, n_chunks=n_chunks, chunk=chunk),
        grid=(b,),
        in_specs=[
            pl.BlockSpec((1, _DP, n), lambda i: (i, 0, 0)),
            pl.BlockSpec((1, _K, _DP), lambda i: (i, 0, 0)),
            pl.BlockSpec(memory_space=pltpu.MemorySpace.HBM),
        ],
        out_specs=pl.BlockSpec((1, 1, n), lambda i: (i, 0, 0)),
        out_shape=jax.ShapeDtypeStruct((b, 1, n), jnp.int32),
        scratch_shapes=[
            pltpu.MemorySpace.VMEM((6 * _DP, n), jnp.bfloat16),
            pltpu.MemorySpace.VMEM((n, _DP), jnp.float32),
            pltpu.SemaphoreType.DMA,
        ],
        compiler_params=pltpu.CompilerParams(
            dimension_semantics=("parallel",)),
    )(xt, c0, xnk)

    return labels.reshape(b, h, w)


# speedup vs baseline: 5.7148x; 1.0672x over previous
import functools

import jax
import jax.numpy as jnp
from jax.experimental import pallas as pl
from jax.experimental.pallas import tpu as pltpu

_K = 64
_ITERS = 10
_D = 5
_DP = 8


def _kmeans_body(xt_ref, c0_ref, xnk_hbm, out_ref, xcat_scr, xnk_scr, dma_sem,
                 *, n_chunks, chunk):
    copy = pltpu.make_async_copy(xnk_hbm.at[pl.program_id(0)], xnk_scr,
                                 dma_sem)
    copy.start()

    def build_chunk(i, _):
        sl = pl.ds(i * chunk, chunk)
        x_ch = xt_ref[0, :, sl]
        hi = x_ch.astype(jnp.bfloat16)
        r1 = x_ch - hi.astype(jnp.float32)
        mid = r1.astype(jnp.bfloat16)
        lo = (r1 - mid.astype(jnp.float32)).astype(jnp.bfloat16)
        xcat_scr[pl.ds(0, _DP), sl] = hi
        xcat_scr[pl.ds(_DP, _DP), sl] = mid
        xcat_scr[pl.ds(2 * _DP, _DP), sl] = hi
        xcat_scr[pl.ds(3 * _DP, _DP), sl] = lo
        xcat_scr[pl.ds(4 * _DP, _DP), sl] = mid
        xcat_scr[pl.ds(5 * _DP, _DP), sl] = hi
        return 0

    jax.lax.fori_loop(0, n_chunks, build_chunk, 0)

    feat_mask = (jax.lax.broadcasted_iota(jnp.int32, (1, _DP), 1) < _D
                 ).astype(jnp.float32)
    col5 = (jax.lax.broadcasted_iota(jnp.int32, (1, _DP), 1) == _D
            ).astype(jnp.float32)

    def make_ccat(c):
        cnorm = jnp.sum(c * c, axis=1, keepdims=True)
        eq3 = c[:, None, :] == c[None, :, :]
        alld = jnp.all(eq3, axis=2)
        jlt = (jax.lax.broadcasted_iota(jnp.int32, (_K, _K), 1)
               < jax.lax.broadcasted_iota(jnp.int32, (_K, _K), 0))
        dupflag = jnp.any(alld & jlt, axis=1, keepdims=True).astype(
            jnp.float32)
        c_aug = -2.0 * c + (cnorm + dupflag * 1e30) * col5
        hi = c_aug.astype(jnp.bfloat16)
        r1 = c_aug - hi.astype(jnp.float32)
        mid = r1.astype(jnp.bfloat16)
        lo = (r1 - mid.astype(jnp.float32)).astype(jnp.bfloat16)
        return jnp.concatenate([hi, hi, mid, hi, mid, lo], axis=1)

    def labels_for_chunk(c_cat, i):
        x_ch = xcat_scr[:, pl.ds(i * chunk, chunk)]
        scores = jax.lax.dot_general(
            c_cat, x_ch, (((1,), (0,)), ((), ())),
            preferred_element_type=jnp.float32)
        mins = jnp.min(scores, axis=0, keepdims=True)
        return scores, mins

    def update_iter(_, c):
        c_cat = make_ccat(c)

        sums = jnp.zeros((_K, _DP), jnp.float32)
        for i in range(n_chunks):
            scores, mins = labels_for_chunk(c_cat, i)
            oh = (scores == mins).astype(jnp.float32)
            xnk_ch = xnk_scr[pl.ds(i * chunk, chunk), :]
            sums = sums + jax.lax.dot_general(
                oh, xnk_ch, (((1,), (0,)), ((), ())),
                preferred_element_type=jnp.float32)
        counts = sums[:, _D:_D + 1]
        return (sums / jnp.maximum(counts, 1.0)) * feat_mask

    copy.wait()
    c = jax.lax.fori_loop(0, _ITERS, update_iter, c0_ref[0])

    c_cat = make_ccat(c)

    for i in range(n_chunks):
        scores, mins = labels_for_chunk(c_cat, i)
        kiota = jax.lax.broadcasted_iota(jnp.int32, (_K, chunk), 0)
        lab = jnp.min(jnp.where(scores == mins, kiota, _K),
                      axis=0, keepdims=True)
        out_ref[0, :, pl.ds(i * chunk, chunk)] = lab


@jax.jit
def kernel(x):
    x = x.astype(jnp.float32) / 255.0
    x = 2.0 * x - 1.0
    b, c, h, w = x.shape
    n = h * w
    ys, xs = jnp.meshgrid(jnp.arange(h, dtype=jnp.float32),
                          jnp.arange(w, dtype=jnp.float32), indexing='ij')
    xgrid = 2.0 * xs / (w - 1) - 1.0
    ygrid = 2.0 * ys / (h - 1) - 1.0
    coords = jnp.broadcast_to(jnp.stack([ygrid, xgrid], axis=0)[None],
                              (b, 2, h, w))
    feats = jnp.concatenate([x, coords], axis=1).reshape(b, _D, n)

    xt = jnp.concatenate([feats,
                          jnp.ones((b, 1, n), jnp.float32),
                          jnp.zeros((b, 2, n), jnp.float32)], axis=1)
    xnk = xt.transpose(0, 2, 1)

    k_inds = jax.random.randint(jax.random.key(1), (b, _K), 0, n)
    c0 = jnp.take_along_axis(feats, k_inds[:, None, :], axis=2)
    c0 = c0.transpose(0, 2, 1)
    c0 = jnp.concatenate([c0, jnp.zeros((b, _K, _DP - _D), jnp.float32)],
                         axis=2)

    n_chunks = 8
    chunk = n // n_chunks

    labels = pl.pallas_call(
        functools.partial(_kmeans_body, n_chunks=n_chunks, chunk=chunk),
        grid=(b,),
        in_specs=[
            pl.BlockSpec((1, _DP, n), lambda i: (i, 0, 0)),
            pl.BlockSpec((1, _K, _DP), lambda i: (i, 0, 0)),
            pl.BlockSpec(memory_space=pltpu.MemorySpace.HBM),
        ],
        out_specs=pl.BlockSpec((1, 1, n), lambda i: (i, 0, 0)),
        out_shape=jax.ShapeDtypeStruct((b, 1, n), jnp.int32),
        scratch_shapes=[
            pltpu.MemorySpace.VMEM((6 * _DP, n), jnp.bfloat16),
            pltpu.MemorySpace.VMEM((n, _DP), jnp.float32),
            pltpu.SemaphoreType.DMA,
        ],
        compiler_params=pltpu.CompilerParams(
            dimension_semantics=("parallel",)),
    )(xt, c0, xnk)

    return labels.reshape(b, h, w)


# speedup vs baseline: 8.7065x; 1.5235x over previous
import functools

import jax
import jax.numpy as jnp
from jax.experimental import pallas as pl
from jax.experimental.pallas import tpu as pltpu

_K = 64
_ITERS = 10
_D = 5
_DP = 8


def _kmeans_body(xn_ref, cy_ref, c0_ref, out_ref, xcat_scr, xnk_scr,
                 *, n_chunks, chunk):
    def build_chunk(i, _):
        sl = pl.ds(i * chunk, chunk)
        x_ch = jnp.concatenate([xn_ref[0, :, sl], cy_ref[0, :_D, sl]],
                               axis=0)
        hi = x_ch.astype(jnp.bfloat16)
        r1 = x_ch - hi.astype(jnp.float32)
        mid = r1.astype(jnp.bfloat16)
        lo = (r1 - mid.astype(jnp.float32)).astype(jnp.bfloat16)
        xcat_scr[pl.ds(0, _DP), sl] = hi
        xcat_scr[pl.ds(_DP, _DP), sl] = mid
        xcat_scr[pl.ds(2 * _DP, _DP), sl] = hi
        xcat_scr[pl.ds(3 * _DP, _DP), sl] = lo
        xcat_scr[pl.ds(4 * _DP, _DP), sl] = mid
        xcat_scr[pl.ds(5 * _DP, _DP), sl] = hi
        xnk_scr[sl, :] = x_ch.T
        return 0

    jax.lax.fori_loop(0, n_chunks, build_chunk, 0)

    feat_mask = (jax.lax.broadcasted_iota(jnp.int32, (1, _DP), 1) < _D
                 ).astype(jnp.float32)
    col5 = (jax.lax.broadcasted_iota(jnp.int32, (1, _DP), 1) == _D
            ).astype(jnp.float32)
    eye8 = (jax.lax.broadcasted_iota(jnp.int32, (_DP, _DP), 0)
            == jax.lax.broadcasted_iota(jnp.int32, (_DP, _DP), 1)
            ).astype(jnp.float32)

    def make_ccat(c):
        cnorm = jnp.sum(c * c, axis=1, keepdims=True)
        ct = jax.lax.dot_general(eye8, c, (((1,), (1,)), ((), ())),
                                 preferred_element_type=jnp.float32)
        dup = None
        for d in range(_DP):
            eqd = c[:, d:d + 1] == ct[d:d + 1, :]
            dup = eqd if dup is None else (dup & eqd)
        jlt = (jax.lax.broadcasted_iota(jnp.int32, (_K, _K), 1)
               < jax.lax.broadcasted_iota(jnp.int32, (_K, _K), 0))
        dupflag = jnp.any(dup & jlt, axis=1, keepdims=True).astype(
            jnp.float32)
        c_aug = -2.0 * c + (cnorm + dupflag * 1e30) * col5
        hi = c_aug.astype(jnp.bfloat16)
        r1 = c_aug - hi.astype(jnp.float32)
        mid = r1.astype(jnp.bfloat16)
        lo = (r1 - mid.astype(jnp.float32)).astype(jnp.bfloat16)
        return jnp.concatenate([hi, hi, mid, hi, mid, lo], axis=1)

    def labels_for_chunk(c_cat, i):
        x_ch = xcat_scr[:, pl.ds(i * chunk, chunk)]
        scores = jax.lax.dot_general(
            c_cat, x_ch, (((1,), (0,)), ((), ())),
            preferred_element_type=jnp.float32)
        mins = jnp.min(scores, axis=0, keepdims=True)
        return scores, mins

    def update_iter(_, c):
        c_cat = make_ccat(c)

        sums = jnp.zeros((_K, _DP), jnp.float32)
        for i in range(n_chunks):
            scores, mins = labels_for_chunk(c_cat, i)
            oh = (scores == mins).astype(jnp.float32)
            xnk_ch = xnk_scr[pl.ds(i * chunk, chunk), :]
            sums = sums + jax.lax.dot_general(
                oh, xnk_ch, (((1,), (0,)), ((), ())),
                preferred_element_type=jnp.float32)
        counts = sums[:, _D:_D + 1]
        return (sums / jnp.maximum(counts, 1.0)) * feat_mask

    c = jax.lax.fori_loop(0, _ITERS, update_iter, c0_ref[0])

    c_cat = make_ccat(c)

    for i in range(n_chunks):
        scores, mins = labels_for_chunk(c_cat, i)
        kiota = jax.lax.broadcasted_iota(jnp.int32, (_K, chunk), 0)
        lab = jnp.min(jnp.where(scores == mins, kiota, _K),
                      axis=0, keepdims=True)
        out_ref[0, :, pl.ds(i * chunk, chunk)] = lab


@jax.jit
def kernel(x):
    x = x.astype(jnp.float32) / 255.0
    x = 2.0 * x - 1.0
    b, c, h, w = x.shape
    n = h * w
    xn = x.reshape(b, c, n)

    ys, xs = jnp.meshgrid(jnp.arange(h, dtype=jnp.float32),
                          jnp.arange(w, dtype=jnp.float32), indexing='ij')
    xgrid = (2.0 * xs / (w - 1) - 1.0).reshape(1, n)
    ygrid = (2.0 * ys / (h - 1) - 1.0).reshape(1, n)
    cyx = jnp.concatenate([ygrid, xgrid, jnp.ones((1, n), jnp.float32),
                           jnp.zeros((_DP - 3, n), jnp.float32)],
                          axis=0)[None]

    k_inds = jax.random.randint(jax.random.key(1), (b, _K), 0, n)
    c0rgb = jnp.take_along_axis(xn, k_inds[:, None, :], axis=2)
    c0yx = cyx[0, :2, :][None]
    c0yx = jnp.take_along_axis(jnp.broadcast_to(c0yx, (b, 2, n)),
                               k_inds[:, None, :], axis=2)
    c0 = jnp.concatenate([c0rgb, c0yx], axis=1).transpose(0, 2, 1)
    c0 = jnp.concatenate([c0, jnp.zeros((b, _K, _DP - _D), jnp.float32)],
                         axis=2)

    n_chunks = 8
    chunk = n // n_chunks

    labels = pl.pallas_call(
        functools.partial(_kmeans_body, n_chunks=n_chunks, chunk=chunk),
        grid=(b,),
        in_specs=[
            pl.BlockSpec((1, 3, n), lambda i: (i, 0, 0)),
            pl.BlockSpec((1, _DP, n), lambda i: (0, 0, 0)),
            pl.BlockSpec((1, _K, _DP), lambda i: (i, 0, 0)),
        ],
        out_specs=pl.BlockSpec((1, 1, n), lambda i: (i, 0, 0)),
        out_shape=jax.ShapeDtypeStruct((b, 1, n), jnp.int32),
        scratch_shapes=[
            pltpu.MemorySpace.VMEM((6 * _DP, n), jnp.bfloat16),
            pltpu.MemorySpace.VMEM((n, _DP), jnp.float32),
        ],
        compiler_params=pltpu.CompilerParams(
            dimension_semantics=("arbitrary",)),
    )(xn, cyx, c0)

    return labels.reshape(b, h, w)


# speedup vs baseline: 12.9503x; 1.4874x over previous
import functools

import jax
import jax.numpy as jnp
from jax.experimental import pallas as pl
from jax.experimental.pallas import tpu as pltpu

_K = 64
_ITERS = 10
_D = 5
_DP = 8


def _kmeans_body(xn_ref, cy_ref, ki_ref, out_ref, xcat_scr, xnk_scr, c0_scr,
                 *, n_chunks, chunk):
    def build_chunk(i, _):
        sl = pl.ds(i * chunk, chunk)
        x_ch = jnp.concatenate([xn_ref[0, :, sl], cy_ref[0, :_D, sl]],
                               axis=0)
        hi = x_ch.astype(jnp.bfloat16)
        r1 = x_ch - hi.astype(jnp.float32)
        mid = r1.astype(jnp.bfloat16)
        lo = (r1 - mid.astype(jnp.float32)).astype(jnp.bfloat16)
        xcat_scr[pl.ds(0, _DP), sl] = hi
        xcat_scr[pl.ds(_DP, _DP), sl] = mid
        xcat_scr[pl.ds(2 * _DP, _DP), sl] = hi
        xcat_scr[pl.ds(3 * _DP, _DP), sl] = lo
        xcat_scr[pl.ds(4 * _DP, _DP), sl] = mid
        xcat_scr[pl.ds(5 * _DP, _DP), sl] = hi
        xnk_scr[sl, :] = x_ch.T
        return 0

    jax.lax.fori_loop(0, n_chunks, build_chunk, 0)

    feat_mask = (jax.lax.broadcasted_iota(jnp.int32, (1, _DP), 1) < _D
                 ).astype(jnp.float32)

    def gather_center(j, _):
        row = xnk_scr[pl.ds(ki_ref[0, 0, j], 1), :]
        c0_scr[pl.ds(j, 1), :] = row * feat_mask
        return 0

    jax.lax.fori_loop(0, _K, gather_center, 0)
    col5 = (jax.lax.broadcasted_iota(jnp.int32, (1, _DP), 1) == _D
            ).astype(jnp.float32)
    eye8 = (jax.lax.broadcasted_iota(jnp.int32, (_DP, _DP), 0)
            == jax.lax.broadcasted_iota(jnp.int32, (_DP, _DP), 1)
            ).astype(jnp.float32)

    def make_ccat(c):
        cnorm = jnp.sum(c * c, axis=1, keepdims=True)
        ct = jax.lax.dot_general(eye8, c, (((1,), (1,)), ((), ())),
                                 preferred_element_type=jnp.float32)
        dup = None
        for d in range(_DP):
            eqd = c[:, d:d + 1] == ct[d:d + 1, :]
            dup = eqd if dup is None else (dup & eqd)
        jlt = (jax.lax.broadcasted_iota(jnp.int32, (_K, _K), 1)
               < jax.lax.broadcasted_iota(jnp.int32, (_K, _K), 0))
        dupflag = jnp.any(dup & jlt, axis=1, keepdims=True).astype(
            jnp.float32)
        c_aug = -2.0 * c + (cnorm + dupflag * 1e30) * col5
        hi = c_aug.astype(jnp.bfloat16)
        r1 = c_aug - hi.astype(jnp.float32)
        mid = r1.astype(jnp.bfloat16)
        lo = (r1 - mid.astype(jnp.float32)).astype(jnp.bfloat16)
        return jnp.concatenate([hi, hi, mid, hi, mid, lo], axis=1)

    def labels_for_chunk(c_cat, i):
        x_ch = xcat_scr[:, pl.ds(i * chunk, chunk)]
        scores = jax.lax.dot_general(
            c_cat, x_ch, (((1,), (0,)), ((), ())),
            preferred_element_type=jnp.float32)
        mins = jnp.min(scores, axis=0, keepdims=True)
        return scores, mins

    def update_iter(_, c):
        c_cat = make_ccat(c)

        sums = jnp.zeros((_K, _DP), jnp.float32)
        for i in range(n_chunks):
            scores, mins = labels_for_chunk(c_cat, i)
            oh = (scores == mins).astype(jnp.float32)
            xnk_ch = xnk_scr[pl.ds(i * chunk, chunk), :]
            sums = sums + jax.lax.dot_general(
                oh, xnk_ch, (((1,), (0,)), ((), ())),
                preferred_element_type=jnp.float32)
        counts = sums[:, _D:_D + 1]
        return (sums / jnp.maximum(counts, 1.0)) * feat_mask

    c = jax.lax.fori_loop(0, _ITERS, update_iter, c0_scr[...])

    c_cat = make_ccat(c)

    for i in range(n_chunks):
        scores, mins = labels_for_chunk(c_cat, i)
        kiota = jax.lax.broadcasted_iota(jnp.int32, (_K, chunk), 0)
        lab = jnp.min(jnp.where(scores == mins, kiota, _K),
                      axis=0, keepdims=True)
        out_ref[0, :, pl.ds(i * chunk, chunk)] = lab


@jax.jit
def kernel(x):
    x = x.astype(jnp.float32) / 255.0
    x = 2.0 * x - 1.0
    b, c, h, w = x.shape
    n = h * w
    xn = x.reshape(b, c, n)

    ys, xs = jnp.meshgrid(jnp.arange(h, dtype=jnp.float32),
                          jnp.arange(w, dtype=jnp.float32), indexing='ij')
    xgrid = (2.0 * xs / (w - 1) - 1.0).reshape(1, n)
    ygrid = (2.0 * ys / (h - 1) - 1.0).reshape(1, n)
    cyx = jnp.concatenate([ygrid, xgrid, jnp.ones((1, n), jnp.float32),
                           jnp.zeros((_DP - 3, n), jnp.float32)],
                          axis=0)[None]

    k_inds = jax.random.randint(jax.random.key(1), (b, _K), 0, n)

    n_chunks = 8
    chunk = n // n_chunks

    labels = pl.pallas_call(
        functools.partial(_kmeans_body, n_chunks=n_chunks, chunk=chunk),
        grid=(b,),
        in_specs=[
            pl.BlockSpec((1, 3, n), lambda i: (i, 0, 0)),
            pl.BlockSpec((1, _DP, n), lambda i: (0, 0, 0)),
            pl.BlockSpec((1, 1, _K), lambda i: (i, 0, 0),
                         memory_space=pltpu.MemorySpace.SMEM),
        ],
        out_specs=pl.BlockSpec((1, 1, n), lambda i: (i, 0, 0)),
        out_shape=jax.ShapeDtypeStruct((b, 1, n), jnp.int32),
        scratch_shapes=[
            pltpu.MemorySpace.VMEM((6 * _DP, n), jnp.bfloat16),
            pltpu.MemorySpace.VMEM((n, _DP), jnp.float32),
            pltpu.MemorySpace.VMEM((_K, _DP), jnp.float32),
        ],
        compiler_params=pltpu.CompilerParams(
            dimension_semantics=("arbitrary",)),
    )(xn, cyx, k_inds[:, None, :])

    return labels.reshape(b, h, w)


# speedup vs baseline: 13.5588x; 1.0470x over previous
import functools

import jax
import jax.numpy as jnp
from jax.experimental import pallas as pl
from jax.experimental.pallas import tpu as pltpu

_K = 64
_ITERS = 10
_D = 5
_DP = 8


def _kmeans_body(xn_ref, cy_ref, ki_ref, out_ref, xcat_scr, xnk_scr, c0_scr,
                 *, n_chunks, chunk):
    def build_chunk(i, _):
        sl = pl.ds(i * chunk, chunk)
        rgb = 2.0 * (xn_ref[0, :, sl] / 255.0) - 1.0
        x_ch = jnp.concatenate([rgb, cy_ref[0, :_D, sl]],
                               axis=0)
        hi = x_ch.astype(jnp.bfloat16)
        r1 = x_ch - hi.astype(jnp.float32)
        mid = r1.astype(jnp.bfloat16)
        lo = (r1 - mid.astype(jnp.float32)).astype(jnp.bfloat16)
        xcat_scr[pl.ds(0, _DP), sl] = hi
        xcat_scr[pl.ds(_DP, _DP), sl] = mid
        xcat_scr[pl.ds(2 * _DP, _DP), sl] = hi
        xcat_scr[pl.ds(3 * _DP, _DP), sl] = lo
        xcat_scr[pl.ds(4 * _DP, _DP), sl] = mid
        xcat_scr[pl.ds(5 * _DP, _DP), sl] = hi
        xnk_scr[sl, :] = x_ch.T
        return 0

    jax.lax.fori_loop(0, n_chunks, build_chunk, 0)

    feat_mask = (jax.lax.broadcasted_iota(jnp.int32, (1, _DP), 1) < _D
                 ).astype(jnp.float32)

    def gather_center(j, _):
        row = xnk_scr[pl.ds(ki_ref[0, 0, j], 1), :]
        c0_scr[pl.ds(j, 1), :] = row * feat_mask
        return 0

    jax.lax.fori_loop(0, _K, gather_center, 0)
    col5 = (jax.lax.broadcasted_iota(jnp.int32, (1, _DP), 1) == _D
            ).astype(jnp.float32)
    eye8 = (jax.lax.broadcasted_iota(jnp.int32, (_DP, _DP), 0)
            == jax.lax.broadcasted_iota(jnp.int32, (_DP, _DP), 1)
            ).astype(jnp.float32)

    def make_ccat(c):
        cnorm = jnp.sum(c * c, axis=1, keepdims=True)
        ct = jax.lax.dot_general(eye8, c, (((1,), (1,)), ((), ())),
                                 preferred_element_type=jnp.float32)
        dup = None
        for d in range(_DP):
            eqd = c[:, d:d + 1] == ct[d:d + 1, :]
            dup = eqd if dup is None else (dup & eqd)
        jlt = (jax.lax.broadcasted_iota(jnp.int32, (_K, _K), 1)
               < jax.lax.broadcasted_iota(jnp.int32, (_K, _K), 0))
        dupflag = jnp.any(dup & jlt, axis=1, keepdims=True).astype(
            jnp.float32)
        c_aug = -2.0 * c + (cnorm + dupflag * 1e30) * col5
        hi = c_aug.astype(jnp.bfloat16)
        r1 = c_aug - hi.astype(jnp.float32)
        mid = r1.astype(jnp.bfloat16)
        lo = (r1 - mid.astype(jnp.float32)).astype(jnp.bfloat16)
        return jnp.concatenate([hi, hi, mid, hi, mid, lo], axis=1)

    def labels_for_chunk(c_cat, i):
        x_ch = xcat_scr[:, pl.ds(i * chunk, chunk)]
        scores = jax.lax.dot_general(
            c_cat, x_ch, (((1,), (0,)), ((), ())),
            preferred_element_type=jnp.float32)
        mins = jnp.min(scores, axis=0, keepdims=True)
        return scores, mins

    def update_iter(_, c):
        c_cat = make_ccat(c)

        sums = jnp.zeros((_K, _DP), jnp.float32)
        for i in range(n_chunks):
            scores, mins = labels_for_chunk(c_cat, i)
            oh = (scores == mins).astype(jnp.float32)
            xnk_ch = xnk_scr[pl.ds(i * chunk, chunk), :]
            sums = sums + jax.lax.dot_general(
                oh, xnk_ch, (((1,), (0,)), ((), ())),
                preferred_element_type=jnp.float32)
        counts = sums[:, _D:_D + 1]
        return (sums / jnp.maximum(counts, 1.0)) * feat_mask

    c = jax.lax.fori_loop(0, _ITERS, update_iter, c0_scr[...])

    c_cat = make_ccat(c)

    for i in range(n_chunks):
        scores, mins = labels_for_chunk(c_cat, i)
        kiota = jax.lax.broadcasted_iota(jnp.int32, (_K, chunk), 0)
        lab = jnp.min(jnp.where(scores == mins, kiota, _K),
                      axis=0, keepdims=True)
        out_ref[0, :, pl.ds(i * chunk, chunk)] = lab


@jax.jit
def kernel(x):
    b, c, h, w = x.shape
    n = h * w
    xn = x.astype(jnp.float32).reshape(b, c, n)

    ys, xs = jnp.meshgrid(jnp.arange(h, dtype=jnp.float32),
                          jnp.arange(w, dtype=jnp.float32), indexing='ij')
    xgrid = (2.0 * xs / (w - 1) - 1.0).reshape(1, n)
    ygrid = (2.0 * ys / (h - 1) - 1.0).reshape(1, n)
    cyx = jnp.concatenate([ygrid, xgrid, jnp.ones((1, n), jnp.float32),
                           jnp.zeros((_DP - 3, n), jnp.float32)],
                          axis=0)[None]

    k_inds = jax.random.randint(jax.random.key(1), (b, _K), 0, n)

    n_chunks = 4
    chunk = n // n_chunks

    labels = pl.pallas_call(
        functools.partial(_kmeans_body, n_chunks=n_chunks, chunk=chunk),
        grid=(b,),
        in_specs=[
            pl.BlockSpec((1, 3, n), lambda i: (i, 0, 0)),
            pl.BlockSpec((1, _DP, n), lambda i: (0, 0, 0)),
            pl.BlockSpec((1, 1, _K), lambda i: (i, 0, 0),
                         memory_space=pltpu.MemorySpace.SMEM),
        ],
        out_specs=pl.BlockSpec((1, 1, n), lambda i: (i, 0, 0)),
        out_shape=jax.ShapeDtypeStruct((b, 1, n), jnp.int32),
        scratch_shapes=[
            pltpu.MemorySpace.VMEM((6 * _DP, n), jnp.bfloat16),
            pltpu.MemorySpace.VMEM((n, _DP), jnp.float32),
            pltpu.MemorySpace.VMEM((_K, _DP), jnp.float32),
        ],
        compiler_params=pltpu.CompilerParams(
            dimension_semantics=("arbitrary",)),
    )(xn, cyx, k_inds[:, None, :])

    return labels.reshape(b, h, w)
